# Initial kernel scaffold; baseline (speedup 1.0000x reference)
import jax
import jax.numpy as jnp
from jax.experimental import pallas as pl


def kernel(words, classes, noun_table, class_table, special_table):
    raise NotImplementedError("write your pallas kernel here")


# speedup vs baseline: 2.5854x; 2.5854x over previous
import functools

import jax
import jax.numpy as jnp
from jax import lax
from jax.experimental import pallas as pl
from jax.experimental.pallas import tpu as pltpu
from jax.experimental.pallas import tpu_sc as plsc

_VOCAB = 1000000
_DIM = 64
_B = 4096
_L = 200

_NC = 2
_NS = 16
_NW = _NC * _NS
_BPW = _B // _NW
_ROW0 = 128
_ROW1 = _L - _ROW0


def _body(words_hbm, classes_hbm, noun_hbm, class_hbm, special_hbm, out_hbm,
          ibufs, cbufs, cls_v, bufs, sem_i0, sem_i1, sem_g0, sem_g1,
          sem_w0, sem_w1):
    wid = lax.axis_index("s") * _NC + lax.axis_index("c")
    base = wid * _BPW

    sem_i = (sem_i0, sem_i1)
    sem_g = (sem_g0, sem_g1)
    sem_w = (sem_w0, sem_w1)

    pltpu.sync_copy(class_hbm, cls_v)
    r0 = [cls_v[0, pl.ds(g * 16, 16)] for g in range(4)]
    r1 = [cls_v[1, pl.ds(g * 16, 16)] for g in range(4)]
    rd = [r1[g] - r0[g] for g in range(4)]

    for p in range(2):
        pltpu.sync_copy(special_hbm, bufs[p].at[pl.ds(0, 1)])

    def fetch_idx(i, p):
        pltpu.async_copy(words_hbm.at[base + i], ibufs[p], sem_i[p])
        pltpu.async_copy(classes_hbm.at[base + i], cbufs[p], sem_i[p])

    def wait_idx(p):
        pltpu.make_async_copy(words_hbm.at[0], ibufs[p], sem_i[p]).wait()
        pltpu.make_async_copy(classes_hbm.at[0], cbufs[p], sem_i[p]).wait()

    def fire_gathers(p):
        pltpu.async_copy(noun_hbm.at[ibufs[p].at[pl.ds(0, _ROW0)]],
                         bufs[p].at[pl.ds(1, _ROW0)], sem_g[p])
        pltpu.async_copy(noun_hbm.at[ibufs[p].at[pl.ds(_ROW0, _ROW1)]],
                         bufs[p].at[pl.ds(1 + _ROW0, _ROW1)], sem_g[p])

    def wait_gathers(p):
        pltpu.make_async_copy(noun_hbm.at[ibufs[p].at[pl.ds(0, _ROW0)]],
                              bufs[p].at[pl.ds(1, _ROW0)], sem_g[p]).wait()
        pltpu.make_async_copy(noun_hbm.at[ibufs[p].at[pl.ds(_ROW0, _ROW1)]],
                              bufs[p].at[pl.ds(1 + _ROW0, _ROW1)],
                              sem_g[p]).wait()

    def fire_write(i, p):
        pltpu.async_copy(bufs[p], out_hbm.at[base + i], sem_w[p])

    def wait_write(p):
        pltpu.make_async_copy(bufs[p], out_hbm.at[0], sem_w[p]).wait()

    def class_add(p):
        buf = bufs[p]
        cbuf = cbufs[p]

        def row_body(l, carry):
            ci = plsc.load_gather(
                cbuf, [jnp.full((16,), l, jnp.int32)]).astype(jnp.float32)
            for g in range(4):
                cur = buf[l + 1, pl.ds(g * 16, 16)]
                buf[l + 1, pl.ds(g * 16, 16)] = cur + r0[g] + ci * rd[g]
            return carry

        lax.fori_loop(0, _L, row_body, 0, unroll=4)

    fetch_idx(0, 0)
    wait_idx(0)
    fire_gathers(0)
    fetch_idx(1, 1)
    wait_idx(1)
    fire_gathers(1)

    def batch_body(i, carry):
        p = i % 2

        def slot_body(p):
            wait_gathers(p)
            class_add(p)
            fire_write(i, p)

            @pl.when(i + 2 < _BPW)
            def _():
                fetch_idx(i + 2, p)
                wait_write(p)
                wait_idx(p)
                fire_gathers(p)

            @pl.when(i + 2 >= _BPW)
            def _():
                wait_write(p)

        lax.cond(p == 0, lambda: slot_body(0), lambda: slot_body(1))
        return carry

    lax.fori_loop(0, _BPW, batch_body, 0)


@jax.jit
def _run(words, classes, noun_table, class_table, special_table):
    mesh = plsc.VectorSubcoreMesh(core_axis_name="c", subcore_axis_name="s")
    kern = pl.kernel(
        _body,
        out_type=jax.ShapeDtypeStruct((_B, _L + 1, _DIM), jnp.float32),
        mesh=mesh,
        compiler_params=pltpu.CompilerParams(needs_layout_passes=False,
                                             use_tc_tiling_on_sc=False),
        scratch_types=[
            [pltpu.VMEM((_L,), jnp.int32) for _ in range(2)],
            [pltpu.VMEM((_L,), jnp.int32) for _ in range(2)],
            pltpu.VMEM((2, _DIM), jnp.float32),
            [pltpu.VMEM((_L + 1, _DIM), jnp.float32) for _ in range(2)],
            pltpu.SemaphoreType.DMA,
            pltpu.SemaphoreType.DMA,
            pltpu.SemaphoreType.DMA,
            pltpu.SemaphoreType.DMA,
            pltpu.SemaphoreType.DMA,
            pltpu.SemaphoreType.DMA,
        ],
    )
    return kern(words, classes, noun_table, class_table, special_table)


def kernel(words, classes, noun_table, class_table, special_table):
    return _run(words.astype(jnp.int32), classes.astype(jnp.int32),
                noun_table, class_table, special_table)
